# Initial kernel scaffold; baseline (speedup 1.0000x reference)
import jax
import jax.numpy as jnp
from jax.experimental import pallas as pl


def kernel(x, dummy):
    raise NotImplementedError("write your pallas kernel here")


# speedup vs baseline: 42.0895x; 42.0895x over previous
import functools

import jax
import jax.numpy as jnp
from jax import lax
from jax.experimental import pallas as pl
from jax.experimental.pallas import tpu as pltpu
from jax.experimental.pallas import tpu_sc as plsc

N = 20000
ROWS = 64
NC, NS = 2, 16
NW = NC * NS
RPW = ROWS // NW
NCHUNK = N // 16
OFF = 16
CMAX = N // 2
PADC = 16
KMAX = 2048
DIST = 10

BIG = 1_000_000
HEIGHT_F = 0.1
PROM_F = 0.05
KEY_HEIGHT = 0x3DCCCCCD
INF_BITS = 0x7F800000


def _key(b):
    return b ^ (lax.shift_right_arithmetic(b, 31) & 0x7FFFFFFF)


def _worker_id():
    return lax.axis_index("s") * NC + lax.axis_index("c")


def _ld(ref, i):
    return ref[pl.ds(i, 16)][0]


def _st(ref, i, val):
    lane = lax.iota(jnp.int32, 16)
    plsc.store_scatter(ref, [jnp.broadcast_to(i, (16,))],
                       jnp.broadcast_to(val, (16,)), mask=lane == 0)


def _peak_body(x_hbm, out_hbm, keyv, outv, posv, valv, stv,
               kposv, kvv, klmv, krmv):
    wid = _worker_id()

    def row_fn(r, _):
        row = wid * RPW + r
        pltpu.sync_copy(x_hbm.at[row], outv)
        keyv[pl.ds(0, 16)] = jnp.full((16,), INF_BITS, jnp.int32)
        keyv[pl.ds(OFF + N, 16)] = jnp.full((16,), INF_BITS, jnp.int32)

        def mm(k, acc):
            mn, mx = acc
            c = outv[pl.ds(k * 16, 16)]
            return jnp.minimum(mn, c), jnp.maximum(mx, c)
        mn16, mx16 = lax.fori_loop(
            0, NCHUNK, mm,
            (jnp.full((16,), jnp.inf, jnp.float32),
             jnp.full((16,), -jnp.inf, jnp.float32)))
        mnv = jnp.broadcast_to(jnp.min(mn16), (16,))
        mxv = jnp.broadcast_to(jnp.max(mx16), (16,))
        dnv = mxv - mnv + jnp.float32(1e-5)

        def nrm(k, _):
            c = outv[pl.ds(k * 16, 16)]
            xnc = (c - mnv) / dnv
            keyv[pl.ds(OFF + k * 16, 16)] = plsc.bitcast(xnc, jnp.int32)
            return 0
        lax.fori_loop(0, NCHUNK, nrm, 0)

        def cand(k, carry):
            cnt, eq = carry
            base = k * 16
            cc = plsc.bitcast(keyv[pl.ds(OFF + base, 16)], jnp.float32)
            ll = plsc.bitcast(keyv[pl.ds(OFF + base - 1, 16)], jnp.float32)
            rr = plsc.bitcast(keyv[pl.ds(OFF + base + 1, 16)], jnp.float32)
            m = (ll < cc) & (rr < cc) & (cc >= HEIGHT_F)
            eq = eq | jnp.int32(jnp.any(cc == rr))
            mi = m.astype(jnp.int32)
            s = jnp.cumsum(mi)
            idx = PADC + cnt + s - 1
            posvec = base + lax.iota(jnp.int32, 16)
            plsc.store_scatter(posv, [idx], posvec, mask=m)
            plsc.store_scatter(valv, [idx],
                               _key(plsc.bitcast(cc, jnp.int32)), mask=m)
            return cnt + jnp.sum(mi), eq
        cnt0, eqany = lax.fori_loop(0, NCHUNK, cand,
                                    (jnp.int32(0), jnp.int32(0)))

        def key_tf(k, _):
            b = keyv[pl.ds(OFF + k * 16, 16)]
            keyv[pl.ds(OFF + k * 16, 16)] = _key(b)
            return 0
        lax.fori_loop(0, NCHUNK, key_tf, 0)

        def rescan():
            def outer_cond(s):
                return s[0] < N - 1
            def outer(s):
                i, cnt = s
                ci = _ld(keyv, OFF + i)
                li = _ld(keyv, OFF + i - 1)
                def rising(cnt):
                    def pcond(t):
                        return t[1] == 0
                    def pbody(t):
                        ia, _ = t
                        ka = _ld(keyv, OFF + ia)
                        stop = (ia >= N - 1) | (ka != ci)
                        return jnp.where(stop, ia, ia + 1), stop.astype(jnp.int32)
                    ia, _ = lax.while_loop(pcond, pbody,
                                           (i + 1, jnp.int32(0)))
                    va = _ld(keyv, OFF + ia)
                    is_peak = va < ci
                    store_it = is_peak & (ci >= KEY_HEIGHT)
                    mid = lax.shift_right_arithmetic(i + ia - 1, 1)
                    @pl.when(store_it)
                    def _():
                        _st(posv, PADC + cnt, mid)
                        _st(valv, PADC + cnt, ci)
                    return (jnp.where(is_peak, ia, i) + 1,
                            cnt + store_it.astype(jnp.int32))
                def flat(cnt):
                    return i + 1, cnt
                i2, cnt2 = lax.cond(li < ci, rising, flat, cnt)
                return i2, cnt2
            _, cntr = lax.while_loop(outer_cond, outer,
                                     (jnp.int32(1), jnp.int32(0)))
            return cntr
        cnt = lax.cond(eqany != 0, rescan, lambda: cnt0)

        posv[pl.ds(0, 16)] = jnp.full((16,), -BIG, jnp.int32)
        stv[pl.ds(0, 16)] = jnp.full((16,), jnp.int32(2), jnp.int32)
        nchc = lax.shift_right_arithmetic(cnt + 15, 4)
        def zst(k, _):
            stv[pl.ds(PADC + k * 16, 16)] = jnp.zeros((16,), jnp.int32)
            return 0
        lax.fori_loop(0, nchc, zst, 0)
        for extra in (0, 16):
            posv[pl.ds(PADC + cnt + extra, 16)] = jnp.full((16,), BIG, jnp.int32)
            stv[pl.ds(PADC + cnt + extra, 16)] = jnp.full((16,), jnp.int32(2), jnp.int32)

        def nms_pass(_):
            def chunk(k, changed):
                base = PADC + k * 16
                pi = posv[pl.ds(base, 16)]
                vi = valv[pl.ds(base, 16)]
                si = stv[pl.ds(base, 16)]
                dom = jnp.zeros((16,), jnp.bool_)
                kn = jnp.zeros((16,), jnp.bool_)
                for d in (1, 2, 3, 4):
                    pL = posv[pl.ds(base - d, 16)]
                    vL = valv[pl.ds(base - d, 16)]
                    sL = stv[pl.ds(base - d, 16)]
                    nearL = (pi - pL) < DIST
                    dom = dom | (nearL & (sL != 2) & (vL > vi))
                    kn = kn | (nearL & (sL == 1))
                    pR = posv[pl.ds(base + d, 16)]
                    vR = valv[pl.ds(base + d, 16)]
                    sR = stv[pl.ds(base + d, 16)]
                    nearR = (pR - pi) < DIST
                    dom = dom | (nearR & (sR != 2) & (vR >= vi))
                    kn = kn | (nearR & (sR == 1))
                und = si == 0
                snew = jnp.where(und & kn, jnp.int32(2),
                                 jnp.where(und & (~dom), jnp.int32(1), si))
                stv[pl.ds(base, 16)] = snew
                return changed | jnp.int32(jnp.any(snew != si))
            return lax.fori_loop(0, nchc, chunk, jnp.int32(0))
        lax.while_loop(lambda ch: ch != 0, nms_pass, jnp.int32(1))

        def z(k, _):
            outv[pl.ds(k * 16, 16)] = jnp.zeros((16,), jnp.float32)
            return 0
        lax.fori_loop(0, NCHUNK, z, 0)

        def prom_body(i, nk):
            st = _ld(stv, PADC + i)
            def kept_fn(nk):
                p = _ld(posv, PADC + i)
                vk = _ld(valv, PADC + i)
                def wcond(s):
                    return s[2] == 0
                def wl(s):
                    j, lm, _ = s
                    kj = _ld(keyv, OFF + j)
                    stop = kj > vk
                    return (j - 1,
                            jnp.where(stop, lm, jnp.minimum(lm, kj)),
                            stop.astype(jnp.int32))
                _, lm, _ = lax.while_loop(wcond, wl, (p, vk, jnp.int32(0)))
                def wr(s):
                    j, rm, _ = s
                    kj = _ld(keyv, OFF + j)
                    stop = kj > vk
                    return (j + 1,
                            jnp.where(stop, rm, jnp.minimum(rm, kj)),
                            stop.astype(jnp.int32))
                _, rm, _ = lax.while_loop(wcond, wr, (p, vk, jnp.int32(0)))
                _st(kposv, nk, p)
                _st(kvv, nk, vk)
                _st(klmv, nk, lm)
                _st(krmv, nk, rm)
                return nk + 1
            return lax.cond(st == 1, kept_fn, lambda nk: nk, nk)
        nk = lax.fori_loop(0, cnt, prom_body, jnp.int32(0))

        nkch = lax.shift_right_arithmetic(nk + 15, 4)
        def fin(k, _):
            base = k * 16
            lane = lax.iota(jnp.int32, 16)
            mask = (base + lane) < nk
            v = plsc.bitcast(_key(kvv[pl.ds(base, 16)]), jnp.float32)
            lm = plsc.bitcast(_key(klmv[pl.ds(base, 16)]), jnp.float32)
            rm = plsc.bitcast(_key(krmv[pl.ds(base, 16)]), jnp.float32)
            ok = mask & ((v - jnp.maximum(lm, rm)) >= PROM_F)
            pp = kposv[pl.ds(base, 16)]
            plsc.store_scatter(outv, [pp],
                               jnp.full((16,), 1.0, jnp.float32), mask=ok)
            return 0
        lax.fori_loop(0, nkch, fin, 0)

        pltpu.sync_copy(outv, out_hbm.at[row])
        return 0

    lax.fori_loop(0, RPW, row_fn, 0)


_mesh = plsc.VectorSubcoreMesh(core_axis_name="c", subcore_axis_name="s",
                               num_cores=NC, num_subcores=NS)

_peaks = pl.kernel(
    _peak_body,
    out_type=jax.ShapeDtypeStruct((ROWS, N), jnp.float32),
    mesh=_mesh,
    compiler_params=pltpu.CompilerParams(needs_layout_passes=False),
    scratch_types=[
        pltpu.VMEM((OFF + N + 16,), jnp.int32),
        pltpu.VMEM((N,), jnp.float32),
        pltpu.VMEM((PADC + CMAX + 32,), jnp.int32),
        pltpu.VMEM((PADC + CMAX + 32,), jnp.int32),
        pltpu.VMEM((PADC + CMAX + 32,), jnp.int32),
        pltpu.VMEM((KMAX,), jnp.int32),
        pltpu.VMEM((KMAX,), jnp.int32),
        pltpu.VMEM((KMAX,), jnp.int32),
        pltpu.VMEM((KMAX,), jnp.int32),
    ],
)


@jax.jit
def kernel(x, dummy):
    del dummy
    return _peaks(x)


# speedup vs baseline: 199.2182x; 4.7332x over previous
import functools

import jax
import jax.numpy as jnp
from jax import lax
from jax.experimental import pallas as pl
from jax.experimental.pallas import tpu as pltpu
from jax.experimental.pallas import tpu_sc as plsc

N = 20000
ROWS = 64
NC, NS = 2, 16
NW = NC * NS
RPW = ROWS // NW
NCHUNK = N // 16
OFF = 16
CMAX = N // 2
PADC = 16
KMAX = 2048
DIST = 10

BIG = 1_000_000
HEIGHT_F = 0.1
PROM_F = 0.05
KEY_HEIGHT = 0x3DCCCCCD
INF_BITS = 0x7F800000


def _key(b):
    return b ^ (lax.shift_right_arithmetic(b, 31) & 0x7FFFFFFF)


def _worker_id():
    return lax.axis_index("s") * NC + lax.axis_index("c")


def _ld(ref, i):
    return ref[pl.ds(i, 16)][0]


def _st(ref, i, val):
    lane = lax.iota(jnp.int32, 16)
    plsc.store_scatter(ref, [jnp.broadcast_to(i, (16,))],
                       jnp.broadcast_to(val, (16,)), mask=lane == 0)


NCH64 = (OFF + N + 16) // 64


def _peak_body(x_hbm, out_hbm, keyv, outv, posv, valv, stv,
               kposv, kvv, cmaxv, cminv):
    wid = _worker_id()

    def row_fn(r, _):
        row = wid * RPW + r
        pltpu.sync_copy(x_hbm.at[row], outv)
        keyv[pl.ds(0, 16)] = jnp.full((16,), INF_BITS, jnp.int32)
        keyv[pl.ds(OFF + N, 16)] = jnp.full((16,), INF_BITS, jnp.int32)

        def mm(k, acc):
            mn, mx = acc
            c = outv[pl.ds(k * 16, 16)]
            return jnp.minimum(mn, c), jnp.maximum(mx, c)
        mn16, mx16 = lax.fori_loop(
            0, NCHUNK, mm,
            (jnp.full((16,), jnp.inf, jnp.float32),
             jnp.full((16,), -jnp.inf, jnp.float32)))
        mnv = jnp.broadcast_to(jnp.min(mn16), (16,))
        mxv = jnp.broadcast_to(jnp.max(mx16), (16,))
        dnv = mxv - mnv + jnp.float32(1e-5)

        def nrm(k, _):
            c = outv[pl.ds(k * 16, 16)]
            xnc = (c - mnv) / dnv
            keyv[pl.ds(OFF + k * 16, 16)] = plsc.bitcast(xnc, jnp.int32)
            return 0
        lax.fori_loop(0, NCHUNK, nrm, 0)

        def cand(k, carry):
            cnt, eq = carry
            base = k * 16
            cc = plsc.bitcast(keyv[pl.ds(OFF + base, 16)], jnp.float32)
            ll = plsc.bitcast(keyv[pl.ds(OFF + base - 1, 16)], jnp.float32)
            rr = plsc.bitcast(keyv[pl.ds(OFF + base + 1, 16)], jnp.float32)
            m = (ll < cc) & (rr < cc) & (cc >= HEIGHT_F)
            eq = eq | jnp.int32(jnp.any(cc == rr))
            mi = m.astype(jnp.int32)
            s = jnp.cumsum(mi)
            idx = PADC + cnt + s - 1
            posvec = base + lax.iota(jnp.int32, 16)
            plsc.store_scatter(posv, [idx], posvec, mask=m)
            plsc.store_scatter(valv, [idx],
                               _key(plsc.bitcast(cc, jnp.int32)), mask=m)
            return cnt + jnp.sum(mi), eq
        cnt0, eqany = lax.fori_loop(0, NCHUNK, cand,
                                    (jnp.int32(0), jnp.int32(0)))

        def key_tf(k, _):
            b = keyv[pl.ds(OFF + k * 16, 16)]
            keyv[pl.ds(OFF + k * 16, 16)] = _key(b)
            return 0
        lax.fori_loop(0, NCHUNK, key_tf, 0)

        def rescan():
            def outer_cond(s):
                return s[0] < N - 1
            def outer(s):
                i, cnt = s
                ci = _ld(keyv, OFF + i)
                li = _ld(keyv, OFF + i - 1)
                def rising(cnt):
                    def pcond(t):
                        return t[1] == 0
                    def pbody(t):
                        ia, _ = t
                        ka = _ld(keyv, OFF + ia)
                        stop = (ia >= N - 1) | (ka != ci)
                        return jnp.where(stop, ia, ia + 1), stop.astype(jnp.int32)
                    ia, _ = lax.while_loop(pcond, pbody,
                                           (i + 1, jnp.int32(0)))
                    va = _ld(keyv, OFF + ia)
                    is_peak = va < ci
                    store_it = is_peak & (ci >= KEY_HEIGHT)
                    mid = lax.shift_right_arithmetic(i + ia - 1, 1)
                    @pl.when(store_it)
                    def _():
                        _st(posv, PADC + cnt, mid)
                        _st(valv, PADC + cnt, ci)
                    return (jnp.where(is_peak, ia, i) + 1,
                            cnt + store_it.astype(jnp.int32))
                def flat(cnt):
                    return i + 1, cnt
                i2, cnt2 = lax.cond(li < ci, rising, flat, cnt)
                return i2, cnt2
            _, cntr = lax.while_loop(outer_cond, outer,
                                     (jnp.int32(1), jnp.int32(0)))
            return cntr
        cnt = lax.cond(eqany != 0, rescan, lambda: cnt0)

        posv[pl.ds(0, 16)] = jnp.full((16,), -BIG, jnp.int32)
        stv[pl.ds(0, 16)] = jnp.full((16,), jnp.int32(2), jnp.int32)
        nchc = lax.shift_right_arithmetic(cnt + 15, 4)
        def zst(k, _):
            stv[pl.ds(PADC + k * 16, 16)] = jnp.zeros((16,), jnp.int32)
            return 0
        lax.fori_loop(0, nchc, zst, 0)
        for extra in (0, 16):
            posv[pl.ds(PADC + cnt + extra, 16)] = jnp.full((16,), BIG, jnp.int32)
            stv[pl.ds(PADC + cnt + extra, 16)] = jnp.full((16,), jnp.int32(2), jnp.int32)

        def nms_pass(_):
            def chunk(k, changed):
                base = PADC + k * 16
                pi = posv[pl.ds(base, 16)]
                vi = valv[pl.ds(base, 16)]
                si = stv[pl.ds(base, 16)]
                dom = jnp.zeros((16,), jnp.bool_)
                kn = jnp.zeros((16,), jnp.bool_)
                for d in (1, 2, 3, 4):
                    pL = posv[pl.ds(base - d, 16)]
                    vL = valv[pl.ds(base - d, 16)]
                    sL = stv[pl.ds(base - d, 16)]
                    nearL = (pi - pL) < DIST
                    dom = dom | (nearL & (sL != 2) & (vL > vi))
                    kn = kn | (nearL & (sL == 1))
                    pR = posv[pl.ds(base + d, 16)]
                    vR = valv[pl.ds(base + d, 16)]
                    sR = stv[pl.ds(base + d, 16)]
                    nearR = (pR - pi) < DIST
                    dom = dom | (nearR & (sR != 2) & (vR >= vi))
                    kn = kn | (nearR & (sR == 1))
                und = si == 0
                snew = jnp.where(und & kn, jnp.int32(2),
                                 jnp.where(und & (~dom), jnp.int32(1), si))
                stv[pl.ds(base, 16)] = snew
                return changed | jnp.int32(jnp.any(snew != si))
            return lax.fori_loop(0, nchc, chunk, jnp.int32(0))
        lax.while_loop(lambda ch: ch != 0, nms_pass, jnp.int32(1))

        def z(k, _):
            outv[pl.ds(k * 16, 16)] = jnp.zeros((16,), jnp.float32)
            return 0
        lax.fori_loop(0, NCHUNK, z, 0)

        def summ(ch, _):
            b = ch * 64
            v0 = keyv[pl.ds(b, 16)]
            v1 = keyv[pl.ds(b + 16, 16)]
            v2 = keyv[pl.ds(b + 32, 16)]
            v3 = keyv[pl.ds(b + 48, 16)]
            hi = jnp.maximum(jnp.maximum(v0, v1), jnp.maximum(v2, v3))
            lo = jnp.minimum(jnp.minimum(v0, v1), jnp.minimum(v2, v3))
            _st(cmaxv, ch, jnp.max(hi))
            _st(cminv, ch, jnp.min(lo))
            return 0
        lax.fori_loop(0, NCH64, summ, 0)

        def keptc(k, nk):
            base = PADC + k * 16
            m = stv[pl.ds(base, 16)] == 1
            mi = m.astype(jnp.int32)
            idx = nk + jnp.cumsum(mi) - 1
            plsc.store_scatter(kposv, [idx], posv[pl.ds(base, 16)], mask=m)
            plsc.store_scatter(kvv, [idx], valv[pl.ds(base, 16)], mask=m)
            return nk + jnp.sum(mi)
        nk = lax.fori_loop(0, nchc, keptc, jnp.int32(0))

        def prom_batch(b, _):
            lane = lax.iota(jnp.int32, 16)
            valid = (b * 16 + lane) < nk
            p = kposv[pl.ds(b * 16, 16)]
            vk = kvv[pl.ds(b * 16, 16)]

            def side(left):
                J0 = OFF + p
                cs = lax.shift_right_arithmetic(J0, 6) * 64
                cedge = cs if left else cs + 63
                mstate = (J0, vk, jnp.zeros((16,), jnp.bool_))

                def acond(s):
                    J, m, stopped = s
                    inch = (J >= cedge) if left else (J <= cedge)
                    return jnp.any(valid & (~stopped) & inch)
                def abody(s):
                    J, m, stopped = s
                    inch = (J >= cedge) if left else (J <= cedge)
                    act = valid & (~stopped) & inch
                    kj = plsc.load_gather(keyv, [J], mask=act)
                    stp = kj > vk
                    m = jnp.where(act & (~stp), jnp.minimum(m, kj), m)
                    stopped = stopped | (act & stp)
                    step = -1 if left else 1
                    J = jnp.where(act & (~stp), J + step, J)
                    return J, m, stopped
                J, m, stopped = lax.while_loop(acond, abody, mstate)

                ch0 = lax.shift_right_arithmetic(J, 6)
                bdone = jnp.zeros((16,), jnp.bool_)
                def bcond(s):
                    ch, m, bd = s
                    return jnp.any(valid & (~stopped) & (~bd))
                def bbody(s):
                    ch, m, bd = s
                    act = valid & (~stopped) & (~bd)
                    cm = plsc.load_gather(cmaxv, [ch], mask=act)
                    cn = plsc.load_gather(cminv, [ch], mask=act)
                    ok = cm <= vk
                    m = jnp.where(act & ok, jnp.minimum(m, cn), m)
                    step = -1 if left else 1
                    ch = jnp.where(act & ok, ch + step, ch)
                    bd = bd | (act & (~ok))
                    return ch, m, bd
                ch, m, _ = lax.while_loop(bcond, bbody, (ch0, m, bdone))

                Jc = ch * 64 + (63 if left else 0)
                def ccond(s):
                    J, m, act = s
                    return jnp.any(act)
                def cbody(s):
                    J, m, act = s
                    kj = plsc.load_gather(keyv, [J], mask=act)
                    stp = kj > vk
                    m = jnp.where(act & (~stp), jnp.minimum(m, kj), m)
                    step = -1 if left else 1
                    J = jnp.where(act & (~stp), J + step, J)
                    return J, m, act & (~stp)
                _, m, _ = lax.while_loop(
                    ccond, cbody, (Jc, m, valid & (~stopped)))
                return m

            lmk = side(True)
            rmk = side(False)
            v = plsc.bitcast(_key(vk), jnp.float32)
            lm = plsc.bitcast(_key(lmk), jnp.float32)
            rm = plsc.bitcast(_key(rmk), jnp.float32)
            ok = valid & ((v - jnp.maximum(lm, rm)) >= PROM_F)
            plsc.store_scatter(outv, [p],
                               jnp.full((16,), 1.0, jnp.float32), mask=ok)
            return 0
        nbat = lax.shift_right_arithmetic(nk + 15, 4)
        lax.fori_loop(0, nbat, prom_batch, 0)

        pltpu.sync_copy(outv, out_hbm.at[row])
        return 0

    lax.fori_loop(0, RPW, row_fn, 0)


_mesh = plsc.VectorSubcoreMesh(core_axis_name="c", subcore_axis_name="s",
                               num_cores=NC, num_subcores=NS)

_peaks = pl.kernel(
    _peak_body,
    out_type=jax.ShapeDtypeStruct((ROWS, N), jnp.float32),
    mesh=_mesh,
    compiler_params=pltpu.CompilerParams(needs_layout_passes=False),
    scratch_types=[
        pltpu.VMEM((OFF + N + 16,), jnp.int32),
        pltpu.VMEM((N,), jnp.float32),
        pltpu.VMEM((PADC + CMAX + 32,), jnp.int32),
        pltpu.VMEM((PADC + CMAX + 32,), jnp.int32),
        pltpu.VMEM((PADC + CMAX + 32,), jnp.int32),
        pltpu.VMEM((KMAX,), jnp.int32),
        pltpu.VMEM((KMAX,), jnp.int32),
        pltpu.VMEM((320,), jnp.int32),
        pltpu.VMEM((320,), jnp.int32),
    ],
)


@jax.jit
def kernel(x, dummy):
    del dummy
    return _peaks(x)


# speedup vs baseline: 256.6569x; 1.2883x over previous
import functools

import jax
import jax.numpy as jnp
from jax import lax
from jax.experimental import pallas as pl
from jax.experimental.pallas import tpu as pltpu
from jax.experimental.pallas import tpu_sc as plsc

N = 20000
ROWS = 64
NC, NS = 2, 16
NW = NC * NS
RPW = ROWS // NW
NCHUNK = N // 16
OFF = 16
CMAX = N // 2
PADC = 16
KMAX = 2048
DIST = 10

BIG = 1_000_000
HEIGHT_F = 0.1
PROM_F = 0.05
KEY_HEIGHT = 0x3DCCCCCD
INF_BITS = 0x7F800000


def _key(b):
    return b ^ (lax.shift_right_arithmetic(b, 31) & 0x7FFFFFFF)


def _worker_id():
    return lax.axis_index("s") * NC + lax.axis_index("c")


def _ld(ref, i):
    return ref[pl.ds(i, 16)][0]


def _st(ref, i, val):
    lane = lax.iota(jnp.int32, 16)
    plsc.store_scatter(ref, [jnp.broadcast_to(i, (16,))],
                       jnp.broadcast_to(val, (16,)), mask=lane == 0)


NCH64 = (OFF + N + 16) // 64


def _peak_body(x_hbm, out_hbm, keyv, outv, posv, valv, stv,
               kposv, kvv, cmaxv, cminv):
    wid = _worker_id()

    def row_fn(r, _):
        row = wid * RPW + r
        pltpu.sync_copy(x_hbm.at[row], outv)
        keyv[pl.ds(0, 16)] = jnp.full((16,), INF_BITS, jnp.int32)
        keyv[pl.ds(OFF + N, 16)] = jnp.full((16,), INF_BITS, jnp.int32)

        def mm(k, acc):
            mn, mx = acc
            c = outv[pl.ds(k * 16, 16)]
            return jnp.minimum(mn, c), jnp.maximum(mx, c)
        mn16, mx16 = lax.fori_loop(
            0, NCHUNK, mm,
            (jnp.full((16,), jnp.inf, jnp.float32),
             jnp.full((16,), -jnp.inf, jnp.float32)), unroll=4)
        mnv = jnp.broadcast_to(jnp.min(mn16), (16,))
        mxv = jnp.broadcast_to(jnp.max(mx16), (16,))
        dnv = mxv - mnv + jnp.float32(1e-5)

        def nrm(k, _):
            c = outv[pl.ds(k * 16, 16)]
            xnc = (c - mnv) / dnv
            keyv[pl.ds(OFF + k * 16, 16)] = plsc.bitcast(xnc, jnp.int32)
            return 0
        lax.fori_loop(0, NCHUNK, nrm, 0, unroll=4)

        def cand(k, carry):
            cnt, eq = carry
            base = k * 16
            cc = plsc.bitcast(keyv[pl.ds(OFF + base, 16)], jnp.float32)
            ll = plsc.bitcast(keyv[pl.ds(OFF + base - 1, 16)], jnp.float32)
            rr = plsc.bitcast(keyv[pl.ds(OFF + base + 1, 16)], jnp.float32)
            m = (ll < cc) & (rr < cc) & (cc >= HEIGHT_F)
            eq = eq | jnp.int32(jnp.any(cc == rr))
            mi = m.astype(jnp.int32)
            s = jnp.cumsum(mi)
            idx = PADC + cnt + s - 1
            posvec = base + lax.iota(jnp.int32, 16)
            plsc.store_scatter(posv, [idx], posvec, mask=m)
            plsc.store_scatter(valv, [idx],
                               _key(plsc.bitcast(cc, jnp.int32)), mask=m)
            return cnt + jnp.sum(mi), eq
        cnt0, eqany = lax.fori_loop(0, NCHUNK, cand,
                                    (jnp.int32(0), jnp.int32(0)), unroll=4)

        def key_tf(k, _):
            b = keyv[pl.ds(OFF + k * 16, 16)]
            keyv[pl.ds(OFF + k * 16, 16)] = _key(b)
            return 0
        lax.fori_loop(0, NCHUNK, key_tf, 0, unroll=4)

        def rescan():
            def outer_cond(s):
                return s[0] < N - 1
            def outer(s):
                i, cnt = s
                ci = _ld(keyv, OFF + i)
                li = _ld(keyv, OFF + i - 1)
                def rising(cnt):
                    def pcond(t):
                        return t[1] == 0
                    def pbody(t):
                        ia, _ = t
                        ka = _ld(keyv, OFF + ia)
                        stop = (ia >= N - 1) | (ka != ci)
                        return jnp.where(stop, ia, ia + 1), stop.astype(jnp.int32)
                    ia, _ = lax.while_loop(pcond, pbody,
                                           (i + 1, jnp.int32(0)))
                    va = _ld(keyv, OFF + ia)
                    is_peak = va < ci
                    store_it = is_peak & (ci >= KEY_HEIGHT)
                    mid = lax.shift_right_arithmetic(i + ia - 1, 1)
                    @pl.when(store_it)
                    def _():
                        _st(posv, PADC + cnt, mid)
                        _st(valv, PADC + cnt, ci)
                    return (jnp.where(is_peak, ia, i) + 1,
                            cnt + store_it.astype(jnp.int32))
                def flat(cnt):
                    return i + 1, cnt
                i2, cnt2 = lax.cond(li < ci, rising, flat, cnt)
                return i2, cnt2
            _, cntr = lax.while_loop(outer_cond, outer,
                                     (jnp.int32(1), jnp.int32(0)))
            return cntr
        cnt = lax.cond(eqany != 0, rescan, lambda: cnt0)

        posv[pl.ds(0, 16)] = jnp.full((16,), -BIG, jnp.int32)
        stv[pl.ds(0, 16)] = jnp.full((16,), jnp.int32(2), jnp.int32)
        nchc = lax.shift_right_arithmetic(cnt + 15, 4)
        def zst(k, _):
            stv[pl.ds(PADC + k * 16, 16)] = jnp.zeros((16,), jnp.int32)
            return 0
        lax.fori_loop(0, nchc, zst, 0)
        for extra in (0, 16):
            posv[pl.ds(PADC + cnt + extra, 16)] = jnp.full((16,), BIG, jnp.int32)
            stv[pl.ds(PADC + cnt + extra, 16)] = jnp.full((16,), jnp.int32(2), jnp.int32)

        def nms_pass(_):
            def chunk(k, changed):
                base = PADC + k * 16
                pi = posv[pl.ds(base, 16)]
                vi = valv[pl.ds(base, 16)]
                si = stv[pl.ds(base, 16)]
                dom = jnp.zeros((16,), jnp.bool_)
                kn = jnp.zeros((16,), jnp.bool_)
                for d in (1, 2, 3, 4):
                    pL = posv[pl.ds(base - d, 16)]
                    vL = valv[pl.ds(base - d, 16)]
                    sL = stv[pl.ds(base - d, 16)]
                    nearL = (pi - pL) < DIST
                    dom = dom | (nearL & (sL != 2) & (vL > vi))
                    kn = kn | (nearL & (sL == 1))
                    pR = posv[pl.ds(base + d, 16)]
                    vR = valv[pl.ds(base + d, 16)]
                    sR = stv[pl.ds(base + d, 16)]
                    nearR = (pR - pi) < DIST
                    dom = dom | (nearR & (sR != 2) & (vR >= vi))
                    kn = kn | (nearR & (sR == 1))
                und = si == 0
                snew = jnp.where(und & kn, jnp.int32(2),
                                 jnp.where(und & (~dom), jnp.int32(1), si))
                stv[pl.ds(base, 16)] = snew
                return changed | jnp.int32(jnp.any(snew != si))
            return lax.fori_loop(0, nchc, chunk, jnp.int32(0))
        lax.while_loop(lambda ch: ch != 0, nms_pass, jnp.int32(1))

        def z(k, _):
            outv[pl.ds(k * 16, 16)] = jnp.zeros((16,), jnp.float32)
            return 0
        lax.fori_loop(0, NCHUNK, z, 0, unroll=4)

        def summ(ch, _):
            b = ch * 64
            v0 = keyv[pl.ds(b, 16)]
            v1 = keyv[pl.ds(b + 16, 16)]
            v2 = keyv[pl.ds(b + 32, 16)]
            v3 = keyv[pl.ds(b + 48, 16)]
            hi = jnp.maximum(jnp.maximum(v0, v1), jnp.maximum(v2, v3))
            lo = jnp.minimum(jnp.minimum(v0, v1), jnp.minimum(v2, v3))
            _st(cmaxv, ch, jnp.max(hi))
            _st(cminv, ch, jnp.min(lo))
            return 0
        lax.fori_loop(0, NCH64, summ, 0, unroll=4)

        def keptc(k, nk):
            base = PADC + k * 16
            m = stv[pl.ds(base, 16)] == 1
            mi = m.astype(jnp.int32)
            idx = nk + jnp.cumsum(mi) - 1
            plsc.store_scatter(kposv, [idx], posv[pl.ds(base, 16)], mask=m)
            plsc.store_scatter(kvv, [idx], valv[pl.ds(base, 16)], mask=m)
            return nk + jnp.sum(mi)
        nk = lax.fori_loop(0, nchc, keptc, jnp.int32(0))

        def prom_batch(b, _):
            lane = lax.iota(jnp.int32, 16)
            valid = (b * 16 + lane) < nk
            p = kposv[pl.ds(b * 16, 16)]
            vk = kvv[pl.ds(b * 16, 16)]

            def side(left):
                J0 = OFF + p
                cs = lax.shift_right_arithmetic(J0, 6) * 64
                cedge = cs if left else cs + 63
                mstate = (J0, vk, jnp.zeros((16,), jnp.bool_))

                def acond(s):
                    J, m, stopped = s
                    inch = (J >= cedge) if left else (J <= cedge)
                    return jnp.any(valid & (~stopped) & inch)
                def abody(s):
                    J, m, stopped = s
                    step = -1 if left else 1
                    for _ in range(4):
                        inch = (J >= cedge) if left else (J <= cedge)
                        act = valid & (~stopped) & inch
                        kj = plsc.load_gather(keyv, [J], mask=act)
                        stp = kj > vk
                        m = jnp.where(act & (~stp), jnp.minimum(m, kj), m)
                        stopped = stopped | (act & stp)
                        J = jnp.where(act & (~stp), J + step, J)
                    return J, m, stopped
                J, m, stopped = lax.while_loop(acond, abody, mstate)

                ch0 = lax.shift_right_arithmetic(J, 6)
                bdone = jnp.zeros((16,), jnp.bool_)
                def bcond(s):
                    ch, m, bd = s
                    return jnp.any(valid & (~stopped) & (~bd))
                def bbody(s):
                    ch, m, bd = s
                    step = -1 if left else 1
                    for _ in range(2):
                        act = valid & (~stopped) & (~bd)
                        cm = plsc.load_gather(cmaxv, [ch], mask=act)
                        cn = plsc.load_gather(cminv, [ch], mask=act)
                        ok = cm <= vk
                        m = jnp.where(act & ok, jnp.minimum(m, cn), m)
                        ch = jnp.where(act & ok, ch + step, ch)
                        bd = bd | (act & (~ok))
                    return ch, m, bd
                ch, m, _ = lax.while_loop(bcond, bbody, (ch0, m, bdone))

                Jc = ch * 64 + (63 if left else 0)
                def ccond(s):
                    J, m, act = s
                    return jnp.any(act)
                def cbody(s):
                    J, m, act = s
                    step = -1 if left else 1
                    for _ in range(4):
                        kj = plsc.load_gather(keyv, [J], mask=act)
                        stp = kj > vk
                        m = jnp.where(act & (~stp), jnp.minimum(m, kj), m)
                        J = jnp.where(act & (~stp), J + step, J)
                        act = act & (~stp)
                    return J, m, act
                _, m, _ = lax.while_loop(
                    ccond, cbody, (Jc, m, valid & (~stopped)))
                return m

            lmk = side(True)
            rmk = side(False)
            v = plsc.bitcast(_key(vk), jnp.float32)
            lm = plsc.bitcast(_key(lmk), jnp.float32)
            rm = plsc.bitcast(_key(rmk), jnp.float32)
            ok = valid & ((v - jnp.maximum(lm, rm)) >= PROM_F)
            plsc.store_scatter(outv, [p],
                               jnp.full((16,), 1.0, jnp.float32), mask=ok)
            return 0
        nbat = lax.shift_right_arithmetic(nk + 15, 4)
        lax.fori_loop(0, nbat, prom_batch, 0)

        pltpu.sync_copy(outv, out_hbm.at[row])
        return 0

    lax.fori_loop(0, RPW, row_fn, 0)


_mesh = plsc.VectorSubcoreMesh(core_axis_name="c", subcore_axis_name="s",
                               num_cores=NC, num_subcores=NS)

_peaks = pl.kernel(
    _peak_body,
    out_type=jax.ShapeDtypeStruct((ROWS, N), jnp.float32),
    mesh=_mesh,
    compiler_params=pltpu.CompilerParams(needs_layout_passes=False),
    scratch_types=[
        pltpu.VMEM((OFF + N + 16,), jnp.int32),
        pltpu.VMEM((N,), jnp.float32),
        pltpu.VMEM((PADC + CMAX + 32,), jnp.int32),
        pltpu.VMEM((PADC + CMAX + 32,), jnp.int32),
        pltpu.VMEM((PADC + CMAX + 32,), jnp.int32),
        pltpu.VMEM((KMAX,), jnp.int32),
        pltpu.VMEM((KMAX,), jnp.int32),
        pltpu.VMEM((320,), jnp.int32),
        pltpu.VMEM((320,), jnp.int32),
    ],
)


@jax.jit
def kernel(x, dummy):
    del dummy
    return _peaks(x)


# speedup vs baseline: 260.3858x; 1.0145x over previous
import functools

import jax
import jax.numpy as jnp
from jax import lax
from jax.experimental import pallas as pl
from jax.experimental.pallas import tpu as pltpu
from jax.experimental.pallas import tpu_sc as plsc

N = 20000
ROWS = 64
NC, NS = 2, 16
NW = NC * NS
RPW = ROWS // NW
NCHUNK = N // 16
OFF = 16
CMAX = N // 2
PADC = 16
KMAX = 2048
DIST = 10

BIG = 1_000_000
HEIGHT_F = 0.1
PROM_F = 0.05
KEY_HEIGHT = 0x3DCCCCCD
INF_BITS = 0x7F800000


def _key(b):
    return b ^ (lax.shift_right_arithmetic(b, 31) & 0x7FFFFFFF)


def _worker_id():
    return lax.axis_index("s") * NC + lax.axis_index("c")


def _ld(ref, i):
    return ref[pl.ds(i, 16)][0]


def _st(ref, i, val):
    lane = lax.iota(jnp.int32, 16)
    plsc.store_scatter(ref, [jnp.broadcast_to(i, (16,))],
                       jnp.broadcast_to(val, (16,)), mask=lane == 0)


NCH64 = (OFF + N + 16) // 64


def _peak_body(x_hbm, out_hbm, keyv, outv, posv, valv, stv,
               kposv, kvv, cmaxv, cminv):
    wid = _worker_id()

    def row_fn(r, _):
        row = wid * RPW + r
        pltpu.sync_copy(x_hbm.at[row], outv)
        keyv[pl.ds(0, 16)] = jnp.full((16,), INF_BITS, jnp.int32)
        keyv[pl.ds(OFF + N, 16)] = jnp.full((16,), INF_BITS, jnp.int32)

        def mm(k, acc):
            mn, mx = acc
            c = outv[pl.ds(k * 16, 16)]
            return jnp.minimum(mn, c), jnp.maximum(mx, c)
        mn16, mx16 = lax.fori_loop(
            0, NCHUNK, mm,
            (jnp.full((16,), jnp.inf, jnp.float32),
             jnp.full((16,), -jnp.inf, jnp.float32)), unroll=4)
        mnv = jnp.broadcast_to(jnp.min(mn16), (16,))
        mxv = jnp.broadcast_to(jnp.max(mx16), (16,))
        dnv = mxv - mnv + jnp.float32(1e-5)

        def nrm(k, _):
            c = outv[pl.ds(k * 16, 16)]
            xnc = (c - mnv) / dnv
            keyv[pl.ds(OFF + k * 16, 16)] = plsc.bitcast(xnc, jnp.int32)
            return 0
        lax.fori_loop(0, NCHUNK, nrm, 0, unroll=4)

        def cand(k, carry):
            cnt, eq = carry
            base = k * 16
            cc = plsc.bitcast(keyv[pl.ds(OFF + base, 16)], jnp.float32)
            ll = plsc.bitcast(keyv[pl.ds(OFF + base - 1, 16)], jnp.float32)
            rr = plsc.bitcast(keyv[pl.ds(OFF + base + 1, 16)], jnp.float32)
            m = (ll < cc) & (rr < cc) & (cc >= HEIGHT_F)
            eq = eq | jnp.int32(jnp.any(cc == rr))
            mi = m.astype(jnp.int32)
            s = jnp.cumsum(mi)
            idx = PADC + cnt + s - 1
            posvec = base + lax.iota(jnp.int32, 16)
            plsc.store_scatter(posv, [idx], posvec, mask=m)
            plsc.store_scatter(valv, [idx],
                               _key(plsc.bitcast(cc, jnp.int32)), mask=m)
            return cnt + jnp.sum(mi), eq
        cnt0, eqany = lax.fori_loop(0, NCHUNK, cand,
                                    (jnp.int32(0), jnp.int32(0)), unroll=4)

        def key_tf(k, _):
            b = keyv[pl.ds(OFF + k * 16, 16)]
            keyv[pl.ds(OFF + k * 16, 16)] = _key(b)
            return 0
        lax.fori_loop(0, NCHUNK, key_tf, 0, unroll=4)

        def rescan():
            def outer_cond(s):
                return s[0] < N - 1
            def outer(s):
                i, cnt = s
                ci = _ld(keyv, OFF + i)
                li = _ld(keyv, OFF + i - 1)
                def rising(cnt):
                    def pcond(t):
                        return t[1] == 0
                    def pbody(t):
                        ia, _ = t
                        ka = _ld(keyv, OFF + ia)
                        stop = (ia >= N - 1) | (ka != ci)
                        return jnp.where(stop, ia, ia + 1), stop.astype(jnp.int32)
                    ia, _ = lax.while_loop(pcond, pbody,
                                           (i + 1, jnp.int32(0)))
                    va = _ld(keyv, OFF + ia)
                    is_peak = va < ci
                    store_it = is_peak & (ci >= KEY_HEIGHT)
                    mid = lax.shift_right_arithmetic(i + ia - 1, 1)
                    @pl.when(store_it)
                    def _():
                        _st(posv, PADC + cnt, mid)
                        _st(valv, PADC + cnt, ci)
                    return (jnp.where(is_peak, ia, i) + 1,
                            cnt + store_it.astype(jnp.int32))
                def flat(cnt):
                    return i + 1, cnt
                i2, cnt2 = lax.cond(li < ci, rising, flat, cnt)
                return i2, cnt2
            _, cntr = lax.while_loop(outer_cond, outer,
                                     (jnp.int32(1), jnp.int32(0)))
            return cntr
        cnt = lax.cond(eqany != 0, rescan, lambda: cnt0)

        posv[pl.ds(0, 16)] = jnp.full((16,), -BIG, jnp.int32)
        stv[pl.ds(0, 16)] = jnp.full((16,), jnp.int32(2), jnp.int32)
        nchc = lax.shift_right_arithmetic(cnt + 15, 4)
        def zst(k, _):
            stv[pl.ds(PADC + k * 16, 16)] = jnp.zeros((16,), jnp.int32)
            return 0
        lax.fori_loop(0, nchc, zst, 0)
        for extra in (0, 16):
            posv[pl.ds(PADC + cnt + extra, 16)] = jnp.full((16,), BIG, jnp.int32)
            stv[pl.ds(PADC + cnt + extra, 16)] = jnp.full((16,), jnp.int32(2), jnp.int32)

        def nms_pass(_):
            def chunk(k, changed):
                base = PADC + k * 16
                si = stv[pl.ds(base, 16)]
                return lax.cond(jnp.any(si == 0), _work, lambda c, *_: c,
                                changed, base, si)
            def _work(changed, base, si):
                pi = posv[pl.ds(base, 16)]
                vi = valv[pl.ds(base, 16)]
                dom = jnp.zeros((16,), jnp.bool_)
                kn = jnp.zeros((16,), jnp.bool_)
                for d in (1, 2, 3, 4):
                    pL = posv[pl.ds(base - d, 16)]
                    vL = valv[pl.ds(base - d, 16)]
                    sL = stv[pl.ds(base - d, 16)]
                    nearL = (pi - pL) < DIST
                    dom = dom | (nearL & (sL != 2) & (vL > vi))
                    kn = kn | (nearL & (sL == 1))
                    pR = posv[pl.ds(base + d, 16)]
                    vR = valv[pl.ds(base + d, 16)]
                    sR = stv[pl.ds(base + d, 16)]
                    nearR = (pR - pi) < DIST
                    dom = dom | (nearR & (sR != 2) & (vR >= vi))
                    kn = kn | (nearR & (sR == 1))
                und = si == 0
                snew = jnp.where(und & kn, jnp.int32(2),
                                 jnp.where(und & (~dom), jnp.int32(1), si))
                stv[pl.ds(base, 16)] = snew
                return changed | jnp.int32(jnp.any(snew != si))
            return lax.fori_loop(0, nchc, chunk, jnp.int32(0))
        lax.while_loop(lambda ch: ch != 0, nms_pass, jnp.int32(1))

        def z(k, _):
            outv[pl.ds(k * 16, 16)] = jnp.zeros((16,), jnp.float32)
            return 0
        lax.fori_loop(0, NCHUNK, z, 0, unroll=4)

        def summ(ch, _):
            b = ch * 64
            v0 = keyv[pl.ds(b, 16)]
            v1 = keyv[pl.ds(b + 16, 16)]
            v2 = keyv[pl.ds(b + 32, 16)]
            v3 = keyv[pl.ds(b + 48, 16)]
            hi = jnp.maximum(jnp.maximum(v0, v1), jnp.maximum(v2, v3))
            lo = jnp.minimum(jnp.minimum(v0, v1), jnp.minimum(v2, v3))
            _st(cmaxv, ch, jnp.max(hi))
            _st(cminv, ch, jnp.min(lo))
            return 0
        lax.fori_loop(0, NCH64, summ, 0, unroll=4)

        def keptc(k, nk):
            base = PADC + k * 16
            m = stv[pl.ds(base, 16)] == 1
            mi = m.astype(jnp.int32)
            idx = nk + jnp.cumsum(mi) - 1
            plsc.store_scatter(kposv, [idx], posv[pl.ds(base, 16)], mask=m)
            plsc.store_scatter(kvv, [idx], valv[pl.ds(base, 16)], mask=m)
            return nk + jnp.sum(mi)
        nk = lax.fori_loop(0, nchc, keptc, jnp.int32(0))

        def prom_batch(b, _):
            lane = lax.iota(jnp.int32, 16)
            valid = (b * 16 + lane) < nk
            p = kposv[pl.ds(b * 16, 16)]
            vk = kvv[pl.ds(b * 16, 16)]

            def side(left):
                J0 = OFF + p
                cs = lax.shift_right_arithmetic(J0, 6) * 64
                cedge = cs if left else cs + 63
                mstate = (J0, vk, jnp.zeros((16,), jnp.bool_))

                def acond(s):
                    J, m, stopped = s
                    inch = (J >= cedge) if left else (J <= cedge)
                    return jnp.any(valid & (~stopped) & inch)
                def abody(s):
                    J, m, stopped = s
                    step = -1 if left else 1
                    for _ in range(4):
                        inch = (J >= cedge) if left else (J <= cedge)
                        act = valid & (~stopped) & inch
                        kj = plsc.load_gather(keyv, [J], mask=act)
                        stp = kj > vk
                        m = jnp.where(act & (~stp), jnp.minimum(m, kj), m)
                        stopped = stopped | (act & stp)
                        J = jnp.where(act & (~stp), J + step, J)
                    return J, m, stopped
                J, m, stopped = lax.while_loop(acond, abody, mstate)

                ch0 = lax.shift_right_arithmetic(J, 6)
                bdone = jnp.zeros((16,), jnp.bool_)
                def bcond(s):
                    ch, m, bd = s
                    return jnp.any(valid & (~stopped) & (~bd))
                def bbody(s):
                    ch, m, bd = s
                    step = -1 if left else 1
                    for _ in range(2):
                        act = valid & (~stopped) & (~bd)
                        cm = plsc.load_gather(cmaxv, [ch], mask=act)
                        cn = plsc.load_gather(cminv, [ch], mask=act)
                        ok = cm <= vk
                        m = jnp.where(act & ok, jnp.minimum(m, cn), m)
                        ch = jnp.where(act & ok, ch + step, ch)
                        bd = bd | (act & (~ok))
                    return ch, m, bd
                ch, m, _ = lax.while_loop(bcond, bbody, (ch0, m, bdone))

                Jc = ch * 64 + (63 if left else 0)
                def ccond(s):
                    J, m, act = s
                    return jnp.any(act)
                def cbody(s):
                    J, m, act = s
                    step = -1 if left else 1
                    for _ in range(4):
                        kj = plsc.load_gather(keyv, [J], mask=act)
                        stp = kj > vk
                        m = jnp.where(act & (~stp), jnp.minimum(m, kj), m)
                        J = jnp.where(act & (~stp), J + step, J)
                        act = act & (~stp)
                    return J, m, act
                _, m, _ = lax.while_loop(
                    ccond, cbody, (Jc, m, valid & (~stopped)))
                return m

            lmk = side(True)
            rmk = side(False)
            v = plsc.bitcast(_key(vk), jnp.float32)
            lm = plsc.bitcast(_key(lmk), jnp.float32)
            rm = plsc.bitcast(_key(rmk), jnp.float32)
            ok = valid & ((v - jnp.maximum(lm, rm)) >= PROM_F)
            plsc.store_scatter(outv, [p],
                               jnp.full((16,), 1.0, jnp.float32), mask=ok)
            return 0
        nbat = lax.shift_right_arithmetic(nk + 15, 4)
        lax.fori_loop(0, nbat, prom_batch, 0)

        pltpu.sync_copy(outv, out_hbm.at[row])
        return 0

    lax.fori_loop(0, RPW, row_fn, 0)


_mesh = plsc.VectorSubcoreMesh(core_axis_name="c", subcore_axis_name="s",
                               num_cores=NC, num_subcores=NS)

_peaks = pl.kernel(
    _peak_body,
    out_type=jax.ShapeDtypeStruct((ROWS, N), jnp.float32),
    mesh=_mesh,
    compiler_params=pltpu.CompilerParams(needs_layout_passes=False),
    scratch_types=[
        pltpu.VMEM((OFF + N + 16,), jnp.int32),
        pltpu.VMEM((N,), jnp.float32),
        pltpu.VMEM((PADC + CMAX + 32,), jnp.int32),
        pltpu.VMEM((PADC + CMAX + 32,), jnp.int32),
        pltpu.VMEM((PADC + CMAX + 32,), jnp.int32),
        pltpu.VMEM((KMAX,), jnp.int32),
        pltpu.VMEM((KMAX,), jnp.int32),
        pltpu.VMEM((320,), jnp.int32),
        pltpu.VMEM((320,), jnp.int32),
    ],
)


@jax.jit
def kernel(x, dummy):
    del dummy
    return _peaks(x)


# speedup vs baseline: 262.6441x; 1.0087x over previous
import functools

import jax
import jax.numpy as jnp
from jax import lax
from jax.experimental import pallas as pl
from jax.experimental.pallas import tpu as pltpu
from jax.experimental.pallas import tpu_sc as plsc

N = 20000
ROWS = 64
NC, NS = 2, 16
NW = NC * NS
RPW = ROWS // NW
NCHUNK = N // 16
OFF = 16
CMAX = N // 2
PADC = 16
KMAX = 2048
DIST = 10

BIG = 1_000_000
HEIGHT_F = 0.1
PROM_F = 0.05
KEY_HEIGHT = 0x3DCCCCCD
INF_BITS = 0x7F800000


def _key(b):
    return b ^ (lax.shift_right_arithmetic(b, 31) & 0x7FFFFFFF)


def _worker_id():
    return lax.axis_index("s") * NC + lax.axis_index("c")


def _ld(ref, i):
    return ref[pl.ds(i, 16)][0]


def _st(ref, i, val):
    lane = lax.iota(jnp.int32, 16)
    plsc.store_scatter(ref, [jnp.broadcast_to(i, (16,))],
                       jnp.broadcast_to(val, (16,)), mask=lane == 0)


NCH64 = (OFF + N + 16) // 64


def _peak_body(x_hbm, out_hbm, keyv, outv, posv, valv, stv,
               kposv, kvv, cmaxv, cminv):
    wid = _worker_id()

    def row_fn(r, _):
        row = wid * RPW + r
        pltpu.sync_copy(x_hbm.at[row], outv)
        keyv[pl.ds(0, 16)] = jnp.full((16,), INF_BITS, jnp.int32)
        keyv[pl.ds(OFF + N, 16)] = jnp.full((16,), INF_BITS, jnp.int32)

        def mm(k, acc):
            mn, mx = acc
            c = outv[pl.ds(k * 16, 16)]
            return jnp.minimum(mn, c), jnp.maximum(mx, c)
        mn16, mx16 = lax.fori_loop(
            0, NCHUNK, mm,
            (jnp.full((16,), jnp.inf, jnp.float32),
             jnp.full((16,), -jnp.inf, jnp.float32)), unroll=4)
        mnv = jnp.broadcast_to(jnp.min(mn16), (16,))
        mxv = jnp.broadcast_to(jnp.max(mx16), (16,))
        dnv = mxv - mnv + jnp.float32(1e-5)

        def nrm(k, _):
            c = outv[pl.ds(k * 16, 16)]
            xnc = (c - mnv) / dnv
            keyv[pl.ds(OFF + k * 16, 16)] = plsc.bitcast(xnc, jnp.int32)
            return 0
        lax.fori_loop(0, NCHUNK, nrm, 0, unroll=4)

        def cand(k, carry):
            cnt, eq = carry
            base = k * 16
            cc = plsc.bitcast(keyv[pl.ds(OFF + base, 16)], jnp.float32)
            ll = plsc.bitcast(keyv[pl.ds(OFF + base - 1, 16)], jnp.float32)
            rr = plsc.bitcast(keyv[pl.ds(OFF + base + 1, 16)], jnp.float32)
            m = (ll < cc) & (rr < cc) & (cc >= HEIGHT_F)
            eq = eq | jnp.int32(jnp.any(cc == rr))
            mi = m.astype(jnp.int32)
            s = jnp.cumsum(mi)
            idx = PADC + cnt + s - 1
            posvec = base + lax.iota(jnp.int32, 16)
            plsc.store_scatter(posv, [idx], posvec, mask=m)
            plsc.store_scatter(valv, [idx],
                               _key(plsc.bitcast(cc, jnp.int32)), mask=m)
            outv[pl.ds(base, 16)] = jnp.zeros((16,), jnp.float32)
            return cnt + jnp.sum(mi), eq
        cnt0, eqany = lax.fori_loop(0, NCHUNK, cand,
                                    (jnp.int32(0), jnp.int32(0)), unroll=4)

        def summ(ch, _):
            b = ch * 64
            vs = []
            for i in range(4):
                v = _key(keyv[pl.ds(b + 16 * i, 16)])
                keyv[pl.ds(b + 16 * i, 16)] = v
                vs.append(v)
            hi = jnp.maximum(jnp.maximum(vs[0], vs[1]),
                             jnp.maximum(vs[2], vs[3]))
            lo = jnp.minimum(jnp.minimum(vs[0], vs[1]),
                             jnp.minimum(vs[2], vs[3]))
            _st(cmaxv, ch, jnp.max(hi))
            _st(cminv, ch, jnp.min(lo))
            return 0
        lax.fori_loop(0, NCH64, summ, 0, unroll=2)

        def rescan():
            def outer_cond(s):
                return s[0] < N - 1
            def outer(s):
                i, cnt = s
                ci = _ld(keyv, OFF + i)
                li = _ld(keyv, OFF + i - 1)
                def rising(cnt):
                    def pcond(t):
                        return t[1] == 0
                    def pbody(t):
                        ia, _ = t
                        ka = _ld(keyv, OFF + ia)
                        stop = (ia >= N - 1) | (ka != ci)
                        return jnp.where(stop, ia, ia + 1), stop.astype(jnp.int32)
                    ia, _ = lax.while_loop(pcond, pbody,
                                           (i + 1, jnp.int32(0)))
                    va = _ld(keyv, OFF + ia)
                    is_peak = va < ci
                    store_it = is_peak & (ci >= KEY_HEIGHT)
                    mid = lax.shift_right_arithmetic(i + ia - 1, 1)
                    @pl.when(store_it)
                    def _():
                        _st(posv, PADC + cnt, mid)
                        _st(valv, PADC + cnt, ci)
                    return (jnp.where(is_peak, ia, i) + 1,
                            cnt + store_it.astype(jnp.int32))
                def flat(cnt):
                    return i + 1, cnt
                i2, cnt2 = lax.cond(li < ci, rising, flat, cnt)
                return i2, cnt2
            _, cntr = lax.while_loop(outer_cond, outer,
                                     (jnp.int32(1), jnp.int32(0)))
            return cntr
        cnt = lax.cond(eqany != 0, rescan, lambda: cnt0)

        posv[pl.ds(0, 16)] = jnp.full((16,), -BIG, jnp.int32)
        stv[pl.ds(0, 16)] = jnp.full((16,), jnp.int32(2), jnp.int32)
        nchc = lax.shift_right_arithmetic(cnt + 15, 4)
        def zst(k, _):
            stv[pl.ds(PADC + k * 16, 16)] = jnp.zeros((16,), jnp.int32)
            return 0
        lax.fori_loop(0, nchc, zst, 0)
        for extra in (0, 16):
            posv[pl.ds(PADC + cnt + extra, 16)] = jnp.full((16,), BIG, jnp.int32)
            stv[pl.ds(PADC + cnt + extra, 16)] = jnp.full((16,), jnp.int32(2), jnp.int32)

        def nms_pass(_):
            def chunk(k, changed):
                base = PADC + k * 16
                si = stv[pl.ds(base, 16)]
                return lax.cond(jnp.any(si == 0), _work, lambda c, *_: c,
                                changed, base, si)
            def _work(changed, base, si):
                pi = posv[pl.ds(base, 16)]
                vi = valv[pl.ds(base, 16)]
                dom = jnp.zeros((16,), jnp.bool_)
                kn = jnp.zeros((16,), jnp.bool_)
                for d in (1, 2, 3, 4):
                    pL = posv[pl.ds(base - d, 16)]
                    vL = valv[pl.ds(base - d, 16)]
                    sL = stv[pl.ds(base - d, 16)]
                    nearL = (pi - pL) < DIST
                    dom = dom | (nearL & (sL != 2) & (vL > vi))
                    kn = kn | (nearL & (sL == 1))
                    pR = posv[pl.ds(base + d, 16)]
                    vR = valv[pl.ds(base + d, 16)]
                    sR = stv[pl.ds(base + d, 16)]
                    nearR = (pR - pi) < DIST
                    dom = dom | (nearR & (sR != 2) & (vR >= vi))
                    kn = kn | (nearR & (sR == 1))
                und = si == 0
                snew = jnp.where(und & kn, jnp.int32(2),
                                 jnp.where(und & (~dom), jnp.int32(1), si))
                stv[pl.ds(base, 16)] = snew
                return changed | jnp.int32(jnp.any(snew != si))
            return lax.fori_loop(0, nchc, chunk, jnp.int32(0))
        lax.while_loop(lambda ch: ch != 0, nms_pass, jnp.int32(1))

        def keptc(k, nk):
            base = PADC + k * 16
            m = stv[pl.ds(base, 16)] == 1
            mi = m.astype(jnp.int32)
            idx = nk + jnp.cumsum(mi) - 1
            plsc.store_scatter(kposv, [idx], posv[pl.ds(base, 16)], mask=m)
            plsc.store_scatter(kvv, [idx], valv[pl.ds(base, 16)], mask=m)
            return nk + jnp.sum(mi)
        nk = lax.fori_loop(0, nchc, keptc, jnp.int32(0))

        def prom_batch(b, _):
            lane = lax.iota(jnp.int32, 16)
            valid = (b * 16 + lane) < nk
            p = kposv[pl.ds(b * 16, 16)]
            vk = kvv[pl.ds(b * 16, 16)]

            def side(left):
                J0 = OFF + p
                cs = lax.shift_right_arithmetic(J0, 6) * 64
                cedge = cs if left else cs + 63
                mstate = (J0, vk, jnp.zeros((16,), jnp.bool_))

                def acond(s):
                    J, m, stopped = s
                    inch = (J >= cedge) if left else (J <= cedge)
                    return jnp.any(valid & (~stopped) & inch)
                def abody(s):
                    J, m, stopped = s
                    step = -1 if left else 1
                    for _ in range(4):
                        inch = (J >= cedge) if left else (J <= cedge)
                        act = valid & (~stopped) & inch
                        kj = plsc.load_gather(keyv, [J], mask=act)
                        stp = kj > vk
                        m = jnp.where(act & (~stp), jnp.minimum(m, kj), m)
                        stopped = stopped | (act & stp)
                        J = jnp.where(act & (~stp), J + step, J)
                    return J, m, stopped
                J, m, stopped = lax.while_loop(acond, abody, mstate)

                ch0 = lax.shift_right_arithmetic(J, 6)
                bdone = jnp.zeros((16,), jnp.bool_)
                def bcond(s):
                    ch, m, bd = s
                    return jnp.any(valid & (~stopped) & (~bd))
                def bbody(s):
                    ch, m, bd = s
                    step = -1 if left else 1
                    for _ in range(2):
                        act = valid & (~stopped) & (~bd)
                        cm = plsc.load_gather(cmaxv, [ch], mask=act)
                        cn = plsc.load_gather(cminv, [ch], mask=act)
                        ok = cm <= vk
                        m = jnp.where(act & ok, jnp.minimum(m, cn), m)
                        ch = jnp.where(act & ok, ch + step, ch)
                        bd = bd | (act & (~ok))
                    return ch, m, bd
                ch, m, _ = lax.while_loop(bcond, bbody, (ch0, m, bdone))

                Jc = ch * 64 + (63 if left else 0)
                def ccond(s):
                    J, m, act = s
                    return jnp.any(act)
                def cbody(s):
                    J, m, act = s
                    step = -1 if left else 1
                    for _ in range(4):
                        kj = plsc.load_gather(keyv, [J], mask=act)
                        stp = kj > vk
                        m = jnp.where(act & (~stp), jnp.minimum(m, kj), m)
                        J = jnp.where(act & (~stp), J + step, J)
                        act = act & (~stp)
                    return J, m, act
                _, m, _ = lax.while_loop(
                    ccond, cbody, (Jc, m, valid & (~stopped)))
                return m

            lmk = side(True)
            rmk = side(False)
            v = plsc.bitcast(_key(vk), jnp.float32)
            lm = plsc.bitcast(_key(lmk), jnp.float32)
            rm = plsc.bitcast(_key(rmk), jnp.float32)
            ok = valid & ((v - jnp.maximum(lm, rm)) >= PROM_F)
            plsc.store_scatter(outv, [p],
                               jnp.full((16,), 1.0, jnp.float32), mask=ok)
            return 0
        nbat = lax.shift_right_arithmetic(nk + 15, 4)
        lax.fori_loop(0, nbat, prom_batch, 0)

        pltpu.sync_copy(outv, out_hbm.at[row])
        return 0

    lax.fori_loop(0, RPW, row_fn, 0)


_mesh = plsc.VectorSubcoreMesh(core_axis_name="c", subcore_axis_name="s",
                               num_cores=NC, num_subcores=NS)

_peaks = pl.kernel(
    _peak_body,
    out_type=jax.ShapeDtypeStruct((ROWS, N), jnp.float32),
    mesh=_mesh,
    compiler_params=pltpu.CompilerParams(needs_layout_passes=False),
    scratch_types=[
        pltpu.VMEM((OFF + N + 16,), jnp.int32),
        pltpu.VMEM((N,), jnp.float32),
        pltpu.VMEM((PADC + CMAX + 32,), jnp.int32),
        pltpu.VMEM((PADC + CMAX + 32,), jnp.int32),
        pltpu.VMEM((PADC + CMAX + 32,), jnp.int32),
        pltpu.VMEM((KMAX,), jnp.int32),
        pltpu.VMEM((KMAX,), jnp.int32),
        pltpu.VMEM((320,), jnp.int32),
        pltpu.VMEM((320,), jnp.int32),
    ],
)


@jax.jit
def kernel(x, dummy):
    del dummy
    return _peaks(x)
